# baseline (speedup 1.0000x reference)
import functools

import jax
import jax.numpy as jnp
from jax import lax
from jax.experimental import pallas as pl
from jax.experimental.pallas import tpu as pltpu
from jax.experimental.pallas import tpu_sc as plsc

CTX = 1000
B, S = 1024, 50
ROWS = B * S
NW = 32
RPW = ROWS // NW
CHUNK = 32
NCHUNK = RPW // CHUNK
PAIRW = 2 * CTX
NPAIR = ROWS // 2
CP = CHUNK // 2
L = 16


NBUF = 4


def _body(seq_hbm, out_hbm, idx_v, buf0, buf1, buf2, buf3,
          sem0, sem1, sem2, sem3):
    cid = lax.axis_index("c")
    sid = lax.axis_index("s")
    wid = sid * 2 + cid
    row0 = wid * RPW

    pltpu.sync_copy(seq_hbm.at[pl.ds(row0, RPW)], idx_v)

    zero16 = jnp.zeros((L,), jnp.float32)
    one16 = jnp.full((L,), 1.0, jnp.float32)
    iota16 = lax.iota(jnp.int32, L)

    bufs = (buf0, buf1, buf2, buf3)
    sems = (sem0, sem1, sem2, sem3)

    def _zero_body(i, carry):
        base = i * L
        for p in range(CP):
            for bz in bufs:
                bz[p, pl.ds(base, L)] = zero16
        return carry
    lax.fori_loop(0, PAIRW // L, _zero_body, 0)

    def _buf_idx(c, o):
        idxs = idx_v[pl.ds(c * CHUNK + o, L)]
        r = iota16 + o
        rows = r >> 1
        cols = (r & 1) * CTX + idxs
        return rows, cols

    pair0 = wid * (RPW // 2)
    handles = [None] * NBUF
    pending = [None] * NBUF
    for c in range(NCHUNK):
        bsel = c % NBUF
        buf = bufs[bsel]
        if handles[bsel] is not None:
            handles[bsel].wait()
            pc = pending[bsel]
            for o in range(0, CHUNK, L):
                rows, cols = _buf_idx(pc, o)
                plsc.store_scatter(buf, [rows, cols], zero16)
        for o in range(0, CHUNK, L):
            rows, cols = _buf_idx(c, o)
            plsc.store_scatter(buf, [rows, cols], one16)
        pairs = iota16 + (pair0 + c * CP)
        handles[bsel] = pltpu.async_copy(buf, out_hbm.at[pairs], sems[bsel])
        pending[bsel] = c
    for h in handles:
        h.wait()


@jax.jit
def _onehot_sc(seq_flat):
    kern = functools.partial(
        pl.kernel,
        mesh=plsc.VectorSubcoreMesh(core_axis_name="c", subcore_axis_name="s"),
        out_type=jax.ShapeDtypeStruct((NPAIR, PAIRW), jnp.float32),
        scratch_types=[
            pltpu.VMEM((RPW,), jnp.int32),
            pltpu.VMEM((CP, PAIRW), jnp.float32),
            pltpu.VMEM((CP, PAIRW), jnp.float32),
            pltpu.VMEM((CP, PAIRW), jnp.float32),
            pltpu.VMEM((CP, PAIRW), jnp.float32),
            pltpu.SemaphoreType.DMA,
            pltpu.SemaphoreType.DMA,
            pltpu.SemaphoreType.DMA,
            pltpu.SemaphoreType.DMA,
        ],
        compiler_params=pltpu.CompilerParams(
            needs_layout_passes=False, use_tc_tiling_on_sc=False),
    )(_body)
    return kern(seq_flat)


def kernel(sequence):
    seq_flat = sequence.reshape(ROWS).astype(jnp.int32)
    out = _onehot_sc(seq_flat)
    return out.reshape(B, S, CTX)
